# Initial kernel scaffold; baseline (speedup 1.0000x reference)
import jax
import jax.numpy as jnp
from jax.experimental import pallas as pl


def kernel(x_reviewer, x_author, edge_index_r2a, edge_index_a2r, W_proj_rev, b_proj_rev, W_proj_aut, b_proj_aut, a_src_r2a, a_dst_r2a, a_src_a2r, a_dst_a2r, q_sem, W_k, b_k, W_fc, b_fc):
    raise NotImplementedError("write your pallas kernel here")


# speedup vs baseline: 1.9805x; 1.9805x over previous
import jax
import jax.numpy as jnp
import numpy as np
from jax.experimental import pallas as pl
from jax.experimental.pallas import tpu as pltpu

_H = 8
_DH = 16
_HID = 128
_EB = 1000


def _proj_kernel(x_ref, w_ref, b_ref, aes_ref, aed_ref, xo_ref, s_ref, d_ref):
    xp = jnp.dot(x_ref[...], w_ref[...], preferred_element_type=jnp.float32)
    xp = xp + b_ref[...]
    xo_ref[...] = xp
    s_ref[...] = jnp.dot(xp, aes_ref[...], preferred_element_type=jnp.float32)
    d_ref[...] = jnp.dot(xp, aed_ref[...], preferred_element_type=jnp.float32)


def _logit_kernel(src_ref, dst_ref, s_ref, d_ref, w_ref):
    def body(i, _):
        s = src_ref[0, 0, i]
        d = dst_ref[0, 0, i]
        z = s_ref[pl.ds(s, 1), :] + d_ref[pl.ds(d, 1), :]
        z = jnp.where(z >= 0, z, 0.2 * z)
        w_ref[pl.ds(i, 1), :] = jnp.exp(z)
        return 0

    jax.lax.fori_loop(0, _EB, body, 0)


def _den_kernel(dst_ref, w_ref, den_ref):
    @pl.when(pl.program_id(0) == 0)
    def _init():
        den_ref[...] = jnp.zeros_like(den_ref)

    def body(i, _):
        d = dst_ref[0, 0, i]
        den_ref[pl.ds(d, 1), :] += w_ref[pl.ds(i, 1), :]
        return 0

    jax.lax.fori_loop(0, _EB, body, 0)


def _num_kernel(src_ref, dst_ref, w_ref, x_ref, num_ref):
    @pl.when(pl.program_id(0) == 0)
    def _init():
        num_ref[...] = jnp.zeros_like(num_ref)

    def body(i, _):
        s = src_ref[0, 0, i]
        d = dst_ref[0, 0, i]
        num_ref[pl.ds(d, 1), :] += w_ref[pl.ds(i, 1), :] * x_ref[pl.ds(s, 1), :]
        return 0

    jax.lax.fori_loop(0, _EB, body, 0)


def _fc_kernel(num_ref, den_ref, wfc_ref, bfc_ref, out_ref):
    feat = jax.nn.relu(num_ref[...] / (den_ref[...] + 1e-16))
    out_ref[...] = jnp.dot(feat, wfc_ref[...],
                           preferred_element_type=jnp.float32) + bfc_ref[...]


def _project(x, w, b, ae_src, ae_dst):
    n = x.shape[0]
    rb = 1000
    shp = jax.ShapeDtypeStruct((n, _HID), jnp.float32)
    return pl.pallas_call(
        _proj_kernel,
        grid=(n // rb,),
        in_specs=[
            pl.BlockSpec((rb, x.shape[1]), lambda i: (i, 0)),
            pl.BlockSpec((x.shape[1], _HID), lambda i: (0, 0)),
            pl.BlockSpec((1, _HID), lambda i: (0, 0)),
            pl.BlockSpec((_HID, _HID), lambda i: (0, 0)),
            pl.BlockSpec((_HID, _HID), lambda i: (0, 0)),
        ],
        out_specs=[pl.BlockSpec((rb, _HID), lambda i: (i, 0))] * 3,
        out_shape=[shp, shp, shp],
    )(x, w, b, ae_src, ae_dst)


def _smem_idx_spec():
    return pl.BlockSpec((1, 1, _EB), lambda b: (b, 0, 0),
                        memory_space=pltpu.SMEM)


def _full_spec(n):
    return pl.BlockSpec((n, _HID), lambda b: (0, 0))


def _edge_pass(src, dst, x_src, s128, d128, n_dst):
    e = src.shape[0]
    nblk = e // _EB
    src3 = src.reshape(nblk, 1, _EB)
    dst3 = dst.reshape(nblk, 1, _EB)
    n_src = x_src.shape[0]
    wblk = pl.BlockSpec((_EB, _HID), lambda b: (b, 0))

    w128 = pl.pallas_call(
        _logit_kernel,
        grid=(nblk,),
        in_specs=[_smem_idx_spec(), _smem_idx_spec(),
                  _full_spec(n_src), _full_spec(n_dst)],
        out_specs=wblk,
        out_shape=jax.ShapeDtypeStruct((e, _HID), jnp.float32),
    )(src3, dst3, s128, d128)

    den = pl.pallas_call(
        _den_kernel,
        grid=(nblk,),
        in_specs=[_smem_idx_spec(), wblk],
        out_specs=_full_spec(n_dst),
        out_shape=jax.ShapeDtypeStruct((n_dst, _HID), jnp.float32),
    )(dst3, w128)

    num = pl.pallas_call(
        _num_kernel,
        grid=(nblk,),
        in_specs=[_smem_idx_spec(), _smem_idx_spec(), wblk, _full_spec(n_src)],
        out_specs=_full_spec(n_dst),
        out_shape=jax.ShapeDtypeStruct((n_dst, _HID), jnp.float32),
    )(src3, dst3, w128, x_src)

    return num, den


def _final_fc(num, den, w_fc, b_fc):
    n = num.shape[0]
    rb = 1000
    out_dim = w_fc.shape[1]
    return pl.pallas_call(
        _fc_kernel,
        grid=(n // rb,),
        in_specs=[
            pl.BlockSpec((rb, _HID), lambda i: (i, 0)),
            pl.BlockSpec((rb, _HID), lambda i: (i, 0)),
            pl.BlockSpec((_HID, out_dim), lambda i: (0, 0)),
            pl.BlockSpec((1, out_dim), lambda i: (0, 0)),
        ],
        out_specs=pl.BlockSpec((rb, out_dim), lambda i: (i, 0)),
        out_shape=jax.ShapeDtypeStruct((n, out_dim), jnp.float32),
    )(num, den, w_fc, b_fc)


_M128 = np.kron(np.eye(_H, dtype=np.float32),
                np.ones((_DH, _DH), dtype=np.float32))


def _ae_mat(a_vec):
    return jnp.asarray(_M128) * a_vec.reshape(-1, 1)


@jax.jit
def kernel(x_reviewer, x_author, edge_index_r2a, edge_index_a2r,
           W_proj_rev, b_proj_rev, W_proj_aut, b_proj_aut,
           a_src_r2a, a_dst_r2a, a_src_a2r, a_dst_a2r,
           q_sem, W_k, b_k, W_fc, b_fc):
    n_rev = x_reviewer.shape[0]
    n_aut = x_author.shape[0]

    xr, s_rev, d_rev = _project(x_reviewer, W_proj_rev,
                                b_proj_rev.reshape(1, -1),
                                _ae_mat(a_src_r2a), _ae_mat(a_dst_a2r))
    xa, s_aut, d_aut = _project(x_author, W_proj_aut,
                                b_proj_aut.reshape(1, -1),
                                _ae_mat(a_src_a2r), _ae_mat(a_dst_r2a))

    num_aut, den_aut = _edge_pass(edge_index_r2a[0], edge_index_r2a[1],
                                  xr, s_rev, d_aut, n_aut)
    num_rev, den_rev = _edge_pass(edge_index_a2r[0], edge_index_a2r[1],
                                  xa, s_aut, d_rev, n_rev)

    num = jnp.concatenate([num_rev, num_aut], axis=0)
    den = jnp.concatenate([den_rev, den_aut], axis=0)
    return _final_fc(num, den, W_fc, b_fc.reshape(1, -1))
